# baseline (speedup 1.0000x reference)
import functools

import jax
import jax.numpy as jnp
from jax import lax
from jax.experimental import pallas as pl
from jax.experimental.pallas import tpu as pltpu
from jax.experimental.pallas import tpu_sc as plsc

M = 512
NW = 32
L = 16


def _row_cumsum_incl(x):
    rows = [x[0:1]]
    for k in range(1, x.shape[0]):
        rows.append(rows[-1] + x[k:k + 1])
    return jnp.concatenate(rows, axis=0)


def _lane_cumsum_incl(x):
    T = x.shape[1]
    acc = x
    s = 1
    while s < T:
        shifted = jnp.concatenate(
            [jnp.zeros_like(acc[:, :s]), acc[:, :-s]], axis=1)
        acc = acc + shifted
        s *= 2
    return acc


def _routing_body(ltT_ref, dst_ref, w_ref, meta_ref):
    ltT = ltT_ref[...]
    E, T = ltT.shape
    mx = jnp.max(ltT, axis=0, keepdims=True)
    p = jnp.exp(ltT - mx)
    probs = p / jnp.sum(p, axis=0, keepdims=True)

    eq0 = (probs == jnp.max(probs, axis=0, keepdims=True)).astype(jnp.float32)
    f0 = eq0 * (_row_cumsum_incl(eq0) == 1.0)
    m0 = jnp.sum(probs * f0, axis=0, keepdims=True)
    pm = jnp.where(f0 > 0, -1.0, probs)
    eq1 = (pm == jnp.max(pm, axis=0, keepdims=True)).astype(jnp.float32)
    f1 = eq1 * (_row_cumsum_incl(eq1) == 1.0)
    m1 = jnp.sum(pm * f1, axis=0, keepdims=True)

    denom = m0 + m1
    w0 = m0 / denom
    w1 = m1 / denom

    c0 = _lane_cumsum_incl(f0)
    c1 = _lane_cumsum_incl(f1)
    cnt0 = c0[:, T - 1:T]
    cnt1 = c1[:, T - 1:T]
    c0ex = c0 - f0
    c1ex = c1 - f1

    cnt = (cnt0 + cnt1).astype(jnp.int32)
    nb = (cnt + (M - 1)) // M
    nb_ex = (_row_cumsum_incl(nb) - nb)
    poff = (nb_ex * M).astype(jnp.float32)

    dst0 = jnp.sum(f0 * (poff + c0ex), axis=0, keepdims=True)
    dst1 = jnp.sum(f1 * (poff + cnt0 + c1ex), axis=0, keepdims=True)

    zi = jnp.zeros((6, T), jnp.int32)
    zf = jnp.zeros((6, T), jnp.float32)
    dst_ref[...] = jnp.concatenate(
        [dst0.astype(jnp.int32), dst1.astype(jnp.int32), zi], axis=0)
    w_ref[...] = jnp.concatenate([w0, w1, zf], axis=0)

    rows_i = lax.broadcasted_iota(jnp.int32, (E, 128), 0)
    lanes_i = lax.broadcasted_iota(jnp.int32, (E, 128), 1)
    sel = (rows_i == lanes_i).astype(jnp.float32)
    nbrow = jnp.sum(nb.astype(jnp.float32) * sel, axis=0, keepdims=True)
    porow = jnp.sum(poff * sel, axis=0, keepdims=True)
    zm = jnp.zeros((6, 128), jnp.float32)
    meta_ref[...] = jnp.concatenate([nbrow, porow, zm], axis=0).astype(jnp.int32)


def _routing(router_logits_T):
    E, T = router_logits_T.shape
    return pl.pallas_call(
        _routing_body,
        out_shape=(
            jax.ShapeDtypeStruct((8, T), jnp.int32),
            jax.ShapeDtypeStruct((8, T), jnp.float32),
            jax.ShapeDtypeStruct((8, 128), jnp.int32),
        ),
    )(router_logits_T)


def _dispatch(hidden_states, dst0r, dst1r, S):
    T, H = hidden_states.shape
    TPW = T // NW
    mesh = plsc.VectorSubcoreMesh(core_axis_name="c", subcore_axis_name="s")

    @functools.partial(
        pl.kernel,
        out_type=jax.ShapeDtypeStruct((S, H), jnp.float32),
        mesh=mesh,
        scratch_types=[
            pltpu.VMEM((TPW, H), jnp.float32),
            pltpu.VMEM((1, TPW), jnp.int32),
            pltpu.VMEM((1, TPW), jnp.int32),
        ],
    )
    def k(x_hbm, d0_hbm, d1_hbm, xs_hbm, xbuf, idx0, idx1):
        wid = lax.axis_index("s") * 2 + lax.axis_index("c")
        base = wid * TPW
        pltpu.sync_copy(d0_hbm.at[pl.ds(wid, 1)], idx0)
        pltpu.sync_copy(d1_hbm.at[pl.ds(wid, 1)], idx1)
        pltpu.sync_copy(x_hbm.at[pl.ds(base, TPW)], xbuf)
        pltpu.sync_copy(xbuf, xs_hbm.at[idx0.at[0]])
        pltpu.sync_copy(xbuf, xs_hbm.at[idx1.at[0]])

    return k(hidden_states, dst0r, dst1r)


def _make_ffn_body(F, ffc, ff):
    def _ffn_body(meta_ref, x_ref, w13_hbm, w2_hbm, y_ref,
                  w1b, w3b, w2b, acc, sems, osem):
        step = pl.program_id(0)
        nsteps = pl.num_programs(0)
        e = step // F
        f = lax.rem(step, F)

        def w_copies(s, sl):
            es = s // F
            fs = lax.rem(s, F)
            return (
                pltpu.make_async_copy(
                    w13_hbm.at[es, pl.ds(fs * ffc, ffc), :],
                    w1b.at[sl], sems.at[sl, 0]),
                pltpu.make_async_copy(
                    w13_hbm.at[es, pl.ds(ff + fs * ffc, ffc), :],
                    w3b.at[sl], sems.at[sl, 1]),
                pltpu.make_async_copy(
                    w2_hbm.at[es, :, pl.ds(fs * ffc, ffc)],
                    w2b.at[sl], sems.at[sl, 2]),
            )

        slot = lax.rem(step, 2)

        @pl.when(step == 0)
        def _():
            for c in w_copies(0, 0):
                c.start()

        @pl.when(step + 1 < nsteps)
        def _():
            for c in w_copies(step + 1, 1 - slot):
                c.start()

        for c in w_copies(step, slot):
            c.wait()

        nb = meta_ref[0, e]
        poff = meta_ref[1, e]
        w1 = w1b[slot].astype(jnp.bfloat16)
        w3 = w3b[slot].astype(jnp.bfloat16)
        w2 = w2b[slot].astype(jnp.bfloat16)
        dn = (((1,), (1,)), ((), ()))

        def blk(i, carry):
            r0 = pl.multiple_of(poff + i * M, M)
            x = x_ref[pl.ds(r0, M), :].astype(jnp.bfloat16)
            gate = lax.dot_general(x, w1, dn,
                                   preferred_element_type=jnp.float32)
            up = lax.dot_general(x, w3, dn,
                                 preferred_element_type=jnp.float32)
            act = (gate * lax.logistic(gate) * up).astype(jnp.bfloat16)
            yv = lax.dot_general(act, w2, dn,
                                 preferred_element_type=jnp.float32)
            a0 = pl.multiple_of(i * M, M)

            @pl.when(f == 0)
            def _():
                acc[pl.ds(a0, M), :] = yv

            @pl.when(f != 0)
            def _():
                acc[pl.ds(a0, M), :] += yv

            @pl.when(f == F - 1)
            def _():
                cp = pltpu.make_async_copy(
                    acc.at[pl.ds(a0, M), :], y_ref.at[pl.ds(r0, M), :], osem)
                cp.start()
                cp.wait()

            return carry

        lax.fori_loop(0, nb, blk, 0)

    return _ffn_body


def _ffn(meta, x_s, w13_weight, w2_weight, F=8):
    S, H = x_s.shape
    E = w13_weight.shape[0]
    ff = w2_weight.shape[2]
    ffc = ff // F
    grid_spec = pltpu.PrefetchScalarGridSpec(
        num_scalar_prefetch=1,
        grid=(E * F,),
        in_specs=[
            pl.BlockSpec((S, H), lambda s, meta: (0, 0)),
            pl.BlockSpec(memory_space=pltpu.MemorySpace.HBM),
            pl.BlockSpec(memory_space=pltpu.MemorySpace.HBM),
        ],
        out_specs=pl.BlockSpec(memory_space=pltpu.MemorySpace.HBM),
        scratch_shapes=[
            pltpu.VMEM((2, ffc, H), jnp.float32),
            pltpu.VMEM((2, ffc, H), jnp.float32),
            pltpu.VMEM((2, H, ffc), jnp.float32),
            pltpu.VMEM((2048, H), jnp.float32),
            pltpu.SemaphoreType.DMA((2, 3)),
            pltpu.SemaphoreType.DMA,
        ],
    )
    return pl.pallas_call(
        _make_ffn_body(F, ffc, ff),
        grid_spec=grid_spec,
        out_shape=jax.ShapeDtypeStruct((S, H), jnp.float32),
        compiler_params=pltpu.CompilerParams(
            dimension_semantics=("arbitrary",),
            vmem_limit_bytes=100 * 1024 * 1024,
        ),
    )(meta, x_s, w13_weight, w2_weight)


def _combine(y_s, d0r, d1r, w0r, w1r, T, H):
    HALF = T // (NW * 2)
    mesh = plsc.VectorSubcoreMesh(core_axis_name="c", subcore_axis_name="s")

    @functools.partial(
        pl.kernel,
        out_type=jax.ShapeDtypeStruct((T, H), jnp.float32),
        mesh=mesh,
        scratch_types=[
            pltpu.VMEM((HALF, H), jnp.float32),
            pltpu.VMEM((HALF, H), jnp.float32),
            pltpu.VMEM((1, HALF), jnp.int32),
            pltpu.VMEM((1, HALF), jnp.int32),
            pltpu.VMEM((1, HALF), jnp.float32),
            pltpu.VMEM((1, HALF), jnp.float32),
            pltpu.SemaphoreType.DMA,
            pltpu.SemaphoreType.DMA,
        ],
    )
    def k(y_hbm, d0_hbm, d1_hbm, w0_hbm, w1_hbm, out_hbm,
          buf0, buf1, idx0, idx1, wb0, wb1, sem0, sem1):
        wid = lax.axis_index("s") * 2 + lax.axis_index("c")

        @pl.loop(0, 2)
        def _(h):
            chunk = wid * 2 + h
            base = chunk * HALF
            pltpu.sync_copy(d0_hbm.at[pl.ds(chunk, 1)], idx0)
            pltpu.sync_copy(d1_hbm.at[pl.ds(chunk, 1)], idx1)
            pltpu.sync_copy(w0_hbm.at[pl.ds(chunk, 1)], wb0)
            pltpu.sync_copy(w1_hbm.at[pl.ds(chunk, 1)], wb1)
            cp0 = pltpu.async_copy(y_hbm.at[idx0.at[0]], buf0, sem0)
            cp1 = pltpu.async_copy(y_hbm.at[idx1.at[0]], buf1, sem1)
            cp0.wait()
            cp1.wait()

            @pl.loop(0, HALF // L)
            def _(g):
                wv0 = wb0[0, pl.ds(g * L, L)]
                wv1 = wb1[0, pl.ds(g * L, L)]

                @pl.loop(0, L)
                def _(j):
                    r = g * L + j
                    jv = jnp.full((L,), j, jnp.int32)
                    w0v = wv0.at[jv].get(mode="promise_in_bounds")
                    w1v = wv1.at[jv].get(mode="promise_in_bounds")

                    @pl.loop(0, H // L)
                    def _(c):
                        a = buf0[r, pl.ds(c * L, L)]
                        b = buf1[r, pl.ds(c * L, L)]
                        buf0[r, pl.ds(c * L, L)] = a * w0v + b * w1v

            pltpu.sync_copy(buf0, out_hbm.at[pl.ds(base, HALF)])

    return k(y_s, d0r, d1r, w0r, w1r)


def kernel(hidden_states, router_logits, w13_weight, w2_weight):
    T, H = hidden_states.shape
    E = router_logits.shape[1]
    K = 2
    S = (T * K // M + E) * M

    dst, w01, meta = _routing(router_logits.T)

    TPW = T // NW
    dst0r = dst[0].reshape(NW, TPW)
    dst1r = dst[1].reshape(NW, TPW)

    x_s = _dispatch(hidden_states, dst0r, dst1r, S)
    y_s = _ffn(meta, x_s, w13_weight, w2_weight)

    HALF = T // (NW * 2)
    d0c = dst[0].reshape(NW * 2, HALF)
    d1c = dst[1].reshape(NW * 2, HALF)
    w0c = w01[0].reshape(NW * 2, HALF)
    w1c = w01[1].reshape(NW * 2, HALF)
    return _combine(y_s, d0c, d1c, w0c, w1c, T, H)


# speedup vs baseline: 1.1052x; 1.1052x over previous
import functools

import jax
import jax.numpy as jnp
from jax import lax
from jax.experimental import pallas as pl
from jax.experimental.pallas import tpu as pltpu
from jax.experimental.pallas import tpu_sc as plsc

M = 512
NW = 32
L = 16


def _row_cumsum_incl(x):
    rows = [x[0:1]]
    for k in range(1, x.shape[0]):
        rows.append(rows[-1] + x[k:k + 1])
    return jnp.concatenate(rows, axis=0)


def _lane_cumsum_incl(x):
    T = x.shape[1]
    acc = x
    s = 1
    while s < T:
        shifted = jnp.concatenate(
            [jnp.zeros_like(acc[:, :s]), acc[:, :-s]], axis=1)
        acc = acc + shifted
        s *= 2
    return acc


def _routing_body(ltT_ref, dst_ref, w_ref, meta_ref):
    ltT = ltT_ref[...]
    E, T = ltT.shape
    mx = jnp.max(ltT, axis=0, keepdims=True)
    p = jnp.exp(ltT - mx)
    probs = p / jnp.sum(p, axis=0, keepdims=True)

    eq0 = (probs == jnp.max(probs, axis=0, keepdims=True)).astype(jnp.float32)
    f0 = eq0 * (_row_cumsum_incl(eq0) == 1.0)
    m0 = jnp.sum(probs * f0, axis=0, keepdims=True)
    pm = jnp.where(f0 > 0, -1.0, probs)
    eq1 = (pm == jnp.max(pm, axis=0, keepdims=True)).astype(jnp.float32)
    f1 = eq1 * (_row_cumsum_incl(eq1) == 1.0)
    m1 = jnp.sum(pm * f1, axis=0, keepdims=True)

    denom = m0 + m1
    w0 = m0 / denom
    w1 = m1 / denom

    c0 = _lane_cumsum_incl(f0)
    c1 = _lane_cumsum_incl(f1)
    cnt0 = c0[:, T - 1:T]
    cnt1 = c1[:, T - 1:T]
    c0ex = c0 - f0
    c1ex = c1 - f1

    cnt = (cnt0 + cnt1).astype(jnp.int32)
    nb = (cnt + (M - 1)) // M
    nb_ex = (_row_cumsum_incl(nb) - nb)
    poff = (nb_ex * M).astype(jnp.float32)

    dst0 = jnp.sum(f0 * (poff + c0ex), axis=0, keepdims=True)
    dst1 = jnp.sum(f1 * (poff + cnt0 + c1ex), axis=0, keepdims=True)

    zi = jnp.zeros((6, T), jnp.int32)
    zf = jnp.zeros((6, T), jnp.float32)
    dst_ref[...] = jnp.concatenate(
        [dst0.astype(jnp.int32), dst1.astype(jnp.int32), zi], axis=0)
    w_ref[...] = jnp.concatenate([w0, w1, zf], axis=0)

    rows_i = lax.broadcasted_iota(jnp.int32, (E, 128), 0)
    lanes_i = lax.broadcasted_iota(jnp.int32, (E, 128), 1)
    sel = (rows_i == lanes_i).astype(jnp.float32)
    nbrow = jnp.sum(nb.astype(jnp.float32) * sel, axis=0, keepdims=True)
    porow = jnp.sum(poff * sel, axis=0, keepdims=True)
    zm = jnp.zeros((6, 128), jnp.float32)
    meta_ref[...] = jnp.concatenate([nbrow, porow, zm], axis=0).astype(jnp.int32)


def _routing(router_logits_T):
    E, T = router_logits_T.shape
    return pl.pallas_call(
        _routing_body,
        out_shape=(
            jax.ShapeDtypeStruct((8, T), jnp.int32),
            jax.ShapeDtypeStruct((8, T), jnp.float32),
            jax.ShapeDtypeStruct((8, 128), jnp.int32),
        ),
    )(router_logits_T)


def _dispatch(hidden_states, dst0r, dst1r, S):
    T, H = hidden_states.shape
    TPW = T // NW
    mesh = plsc.VectorSubcoreMesh(core_axis_name="c", subcore_axis_name="s")

    @functools.partial(
        pl.kernel,
        out_type=jax.ShapeDtypeStruct((S, H), jnp.float32),
        mesh=mesh,
        scratch_types=[
            pltpu.VMEM((TPW, H), jnp.float32),
            pltpu.VMEM((1, TPW), jnp.int32),
            pltpu.VMEM((1, TPW), jnp.int32),
        ],
    )
    def k(x_hbm, d0_hbm, d1_hbm, xs_hbm, xbuf, idx0, idx1):
        wid = lax.axis_index("s") * 2 + lax.axis_index("c")
        base = wid * TPW
        pltpu.sync_copy(d0_hbm.at[pl.ds(wid, 1)], idx0)
        pltpu.sync_copy(d1_hbm.at[pl.ds(wid, 1)], idx1)
        pltpu.sync_copy(x_hbm.at[pl.ds(base, TPW)], xbuf)
        pltpu.sync_copy(xbuf, xs_hbm.at[idx0.at[0]])
        pltpu.sync_copy(xbuf, xs_hbm.at[idx1.at[0]])

    return k(hidden_states, dst0r, dst1r)


def _make_ffn_body(F, ffc, ff):
    def _ffn_body(meta_ref, x_ref, w13_hbm, w2_hbm, y_ref,
                  w1b, w3b, w2b, acc, sems, osem):
        step = pl.program_id(0)
        nsteps = pl.num_programs(0)
        e = step // F
        f = lax.rem(step, F)

        def w_copies(s, sl):
            es = s // F
            fs = lax.rem(s, F)
            return (
                pltpu.make_async_copy(
                    w13_hbm.at[es, pl.ds(fs * ffc, ffc), :],
                    w1b.at[sl], sems.at[sl, 0]),
                pltpu.make_async_copy(
                    w13_hbm.at[es, pl.ds(ff + fs * ffc, ffc), :],
                    w3b.at[sl], sems.at[sl, 1]),
                pltpu.make_async_copy(
                    w2_hbm.at[es, :, pl.ds(fs * ffc, ffc)],
                    w2b.at[sl], sems.at[sl, 2]),
            )

        slot = lax.rem(step, 2)

        @pl.when(step == 0)
        def _():
            for c in w_copies(0, 0):
                c.start()

        @pl.when(step + 1 < nsteps)
        def _():
            for c in w_copies(step + 1, 1 - slot):
                c.start()

        for c in w_copies(step, slot):
            c.wait()

        nb = meta_ref[0, e]
        poff = meta_ref[1, e]
        w1 = w1b[slot].astype(jnp.bfloat16)
        w3 = w3b[slot].astype(jnp.bfloat16)
        w2 = w2b[slot].astype(jnp.bfloat16)
        dn = (((1,), (1,)), ((), ()))

        def blk(i, carry):
            r0 = pl.multiple_of(poff + i * M, M)
            x = x_ref[pl.ds(r0, M), :].astype(jnp.bfloat16)
            gate = lax.dot_general(x, w1, dn,
                                   preferred_element_type=jnp.float32)
            up = lax.dot_general(x, w3, dn,
                                 preferred_element_type=jnp.float32)
            act = (gate * lax.logistic(gate) * up).astype(jnp.bfloat16)
            yv = lax.dot_general(act, w2, dn,
                                 preferred_element_type=jnp.float32)
            a0 = pl.multiple_of(i * M, M)

            @pl.when(f == 0)
            def _():
                acc[pl.ds(a0, M), :] = yv

            @pl.when(f != 0)
            def _():
                acc[pl.ds(a0, M), :] += yv

            @pl.when(f == F - 1)
            def _():
                cp = pltpu.make_async_copy(
                    acc.at[pl.ds(a0, M), :], y_ref.at[pl.ds(r0, M), :], osem)
                cp.start()
                cp.wait()

            return carry

        lax.fori_loop(0, nb, blk, 0)

    return _ffn_body


def _ffn(meta, x_s, w13_weight, w2_weight, F=4):
    S, H = x_s.shape
    E = w13_weight.shape[0]
    ff = w2_weight.shape[2]
    ffc = ff // F
    grid_spec = pltpu.PrefetchScalarGridSpec(
        num_scalar_prefetch=1,
        grid=(E * F,),
        in_specs=[
            pl.BlockSpec((S, H), lambda s, meta: (0, 0)),
            pl.BlockSpec(memory_space=pltpu.MemorySpace.HBM),
            pl.BlockSpec(memory_space=pltpu.MemorySpace.HBM),
        ],
        out_specs=pl.BlockSpec(memory_space=pltpu.MemorySpace.HBM),
        scratch_shapes=[
            pltpu.VMEM((2, ffc, H), jnp.float32),
            pltpu.VMEM((2, ffc, H), jnp.float32),
            pltpu.VMEM((2, H, ffc), jnp.float32),
            pltpu.VMEM((2048, H), jnp.float32),
            pltpu.SemaphoreType.DMA((2, 3)),
            pltpu.SemaphoreType.DMA,
        ],
    )
    return pl.pallas_call(
        _make_ffn_body(F, ffc, ff),
        grid_spec=grid_spec,
        out_shape=jax.ShapeDtypeStruct((S, H), jnp.float32),
        compiler_params=pltpu.CompilerParams(
            dimension_semantics=("arbitrary",),
            vmem_limit_bytes=100 * 1024 * 1024,
        ),
    )(meta, x_s, w13_weight, w2_weight)


def _combine(y_s, d0r, d1r, w0r, w1r, T, H):
    HALF = T // (NW * 2)
    mesh = plsc.VectorSubcoreMesh(core_axis_name="c", subcore_axis_name="s")

    @functools.partial(
        pl.kernel,
        out_type=jax.ShapeDtypeStruct((T, H), jnp.float32),
        mesh=mesh,
        scratch_types=[
            pltpu.VMEM((HALF, H), jnp.float32),
            pltpu.VMEM((HALF, H), jnp.float32),
            pltpu.VMEM((1, HALF), jnp.int32),
            pltpu.VMEM((1, HALF), jnp.int32),
            pltpu.VMEM((1, HALF), jnp.float32),
            pltpu.VMEM((1, HALF), jnp.float32),
            pltpu.SemaphoreType.DMA,
            pltpu.SemaphoreType.DMA,
        ],
    )
    def k(y_hbm, d0_hbm, d1_hbm, w0_hbm, w1_hbm, out_hbm,
          buf0, buf1, idx0, idx1, wb0, wb1, sem0, sem1):
        wid = lax.axis_index("s") * 2 + lax.axis_index("c")

        @pl.loop(0, 2)
        def _(h):
            chunk = wid * 2 + h
            base = chunk * HALF
            pltpu.sync_copy(d0_hbm.at[pl.ds(chunk, 1)], idx0)
            pltpu.sync_copy(d1_hbm.at[pl.ds(chunk, 1)], idx1)
            pltpu.sync_copy(w0_hbm.at[pl.ds(chunk, 1)], wb0)
            pltpu.sync_copy(w1_hbm.at[pl.ds(chunk, 1)], wb1)
            cp0 = pltpu.async_copy(y_hbm.at[idx0.at[0]], buf0, sem0)
            cp1 = pltpu.async_copy(y_hbm.at[idx1.at[0]], buf1, sem1)
            cp0.wait()
            cp1.wait()

            @pl.loop(0, HALF // L)
            def _(g):
                wv0 = wb0[0, pl.ds(g * L, L)]
                wv1 = wb1[0, pl.ds(g * L, L)]

                @pl.loop(0, L)
                def _(j):
                    r = g * L + j
                    jv = jnp.full((L,), j, jnp.int32)
                    w0v = wv0.at[jv].get(mode="promise_in_bounds")
                    w1v = wv1.at[jv].get(mode="promise_in_bounds")

                    @pl.loop(0, H // L)
                    def _(c):
                        a = buf0[r, pl.ds(c * L, L)]
                        b = buf1[r, pl.ds(c * L, L)]
                        buf0[r, pl.ds(c * L, L)] = a * w0v + b * w1v

            pltpu.sync_copy(buf0, out_hbm.at[pl.ds(base, HALF)])

    return k(y_s, d0r, d1r, w0r, w1r)


def kernel(hidden_states, router_logits, w13_weight, w2_weight):
    T, H = hidden_states.shape
    E = router_logits.shape[1]
    K = 2
    S = (T * K // M + E) * M

    dst, w01, meta = _routing(router_logits.T)

    TPW = T // NW
    dst0r = dst[0].reshape(NW, TPW)
    dst1r = dst[1].reshape(NW, TPW)

    x_s = _dispatch(hidden_states, dst0r, dst1r, S)
    y_s = _ffn(meta, x_s, w13_weight, w2_weight)

    HALF = T // (NW * 2)
    d0c = dst[0].reshape(NW * 2, HALF)
    d1c = dst[1].reshape(NW * 2, HALF)
    w0c = w01[0].reshape(NW * 2, HALF)
    w1c = w01[1].reshape(NW * 2, HALF)
    return _combine(y_s, d0c, d1c, w0c, w1c, T, H)


# speedup vs baseline: 1.1888x; 1.0756x over previous
import functools

import jax
import jax.numpy as jnp
from jax import lax
from jax.experimental import pallas as pl
from jax.experimental.pallas import tpu as pltpu
from jax.experimental.pallas import tpu_sc as plsc

M = 512
NW = 32
L = 16


def _row_cumsum_incl(x):
    rows = [x[0:1]]
    for k in range(1, x.shape[0]):
        rows.append(rows[-1] + x[k:k + 1])
    return jnp.concatenate(rows, axis=0)


def _lane_cumsum_incl(x):
    T = x.shape[1]
    acc = x
    s = 1
    while s < T:
        shifted = jnp.concatenate(
            [jnp.zeros_like(acc[:, :s]), acc[:, :-s]], axis=1)
        acc = acc + shifted
        s *= 2
    return acc


def _routing_body(ltT_ref, dst_ref, w_ref, meta_ref):
    ltT = ltT_ref[...]
    E, T = ltT.shape
    mx = jnp.max(ltT, axis=0, keepdims=True)
    p = jnp.exp(ltT - mx)
    probs = p / jnp.sum(p, axis=0, keepdims=True)

    eq0 = (probs == jnp.max(probs, axis=0, keepdims=True)).astype(jnp.float32)
    f0 = eq0 * (_row_cumsum_incl(eq0) == 1.0)
    m0 = jnp.sum(probs * f0, axis=0, keepdims=True)
    pm = jnp.where(f0 > 0, -1.0, probs)
    eq1 = (pm == jnp.max(pm, axis=0, keepdims=True)).astype(jnp.float32)
    f1 = eq1 * (_row_cumsum_incl(eq1) == 1.0)
    m1 = jnp.sum(pm * f1, axis=0, keepdims=True)

    denom = m0 + m1
    w0 = m0 / denom
    w1 = m1 / denom

    c0 = _lane_cumsum_incl(f0)
    c1 = _lane_cumsum_incl(f1)
    cnt0 = c0[:, T - 1:T]
    cnt1 = c1[:, T - 1:T]
    c0ex = c0 - f0
    c1ex = c1 - f1

    cnt = (cnt0 + cnt1).astype(jnp.int32)
    nb = (cnt + (M - 1)) // M
    nb_ex = (_row_cumsum_incl(nb) - nb)
    poff = (nb_ex * M).astype(jnp.float32)

    dst0 = jnp.sum(f0 * (poff + c0ex), axis=0, keepdims=True)
    dst1 = jnp.sum(f1 * (poff + cnt0 + c1ex), axis=0, keepdims=True)

    zi = jnp.zeros((6, T), jnp.int32)
    zf = jnp.zeros((6, T), jnp.float32)
    dst_ref[...] = jnp.concatenate(
        [dst0.astype(jnp.int32), dst1.astype(jnp.int32), zi], axis=0)
    w_ref[...] = jnp.concatenate([w0, w1, zf], axis=0)

    rows_i = lax.broadcasted_iota(jnp.int32, (E, 128), 0)
    lanes_i = lax.broadcasted_iota(jnp.int32, (E, 128), 1)
    sel = (rows_i == lanes_i).astype(jnp.float32)
    nbrow = jnp.sum(nb.astype(jnp.float32) * sel, axis=0, keepdims=True)
    porow = jnp.sum(poff * sel, axis=0, keepdims=True)
    zm = jnp.zeros((6, 128), jnp.float32)
    meta_ref[...] = jnp.concatenate([nbrow, porow, zm], axis=0).astype(jnp.int32)


def _routing(router_logits_T):
    E, T = router_logits_T.shape
    return pl.pallas_call(
        _routing_body,
        out_shape=(
            jax.ShapeDtypeStruct((8, T), jnp.int32),
            jax.ShapeDtypeStruct((8, T), jnp.float32),
            jax.ShapeDtypeStruct((8, 128), jnp.int32),
        ),
    )(router_logits_T)


def _dispatch(hidden_states, dst0r, dst1r, S):
    T, H = hidden_states.shape
    TPW = T // NW
    mesh = plsc.VectorSubcoreMesh(core_axis_name="c", subcore_axis_name="s")

    @functools.partial(
        pl.kernel,
        out_type=jax.ShapeDtypeStruct((S, H), jnp.float32),
        mesh=mesh,
        scratch_types=[
            pltpu.VMEM((TPW, H), jnp.float32),
            pltpu.VMEM((1, TPW), jnp.int32),
            pltpu.VMEM((1, TPW), jnp.int32),
            pltpu.SemaphoreType.DMA,
            pltpu.SemaphoreType.DMA,
        ],
    )
    def k(x_hbm, d0_hbm, d1_hbm, xs_hbm, xbuf, idx0, idx1, ssem0, ssem1):
        wid = lax.axis_index("s") * 2 + lax.axis_index("c")
        base = wid * TPW
        pltpu.sync_copy(d0_hbm.at[pl.ds(wid, 1)], idx0)
        pltpu.sync_copy(d1_hbm.at[pl.ds(wid, 1)], idx1)
        pltpu.sync_copy(x_hbm.at[pl.ds(base, TPW)], xbuf)
        s0 = pltpu.async_copy(xbuf, xs_hbm.at[idx0.at[0]], ssem0)
        s1 = pltpu.async_copy(xbuf, xs_hbm.at[idx1.at[0]], ssem1)
        s0.wait()
        s1.wait()

    return k(hidden_states, dst0r, dst1r)


def _make_ffn_body(F, ffc, ff):
    def _ffn_body(meta_ref, x_ref, w13_hbm, w2_hbm, y_ref,
                  w1b, w3b, w2b, acc, sems, osem):
        step = pl.program_id(0)
        nsteps = pl.num_programs(0)
        e = step // F
        f = lax.rem(step, F)

        def w_copies(s, sl):
            es = s // F
            fs = lax.rem(s, F)
            return (
                pltpu.make_async_copy(
                    w13_hbm.at[es, pl.ds(fs * ffc, ffc), :],
                    w1b.at[sl], sems.at[sl, 0]),
                pltpu.make_async_copy(
                    w13_hbm.at[es, pl.ds(ff + fs * ffc, ffc), :],
                    w3b.at[sl], sems.at[sl, 1]),
                pltpu.make_async_copy(
                    w2_hbm.at[es, :, pl.ds(fs * ffc, ffc)],
                    w2b.at[sl], sems.at[sl, 2]),
            )

        slot = lax.rem(step, 2)

        @pl.when(step == 0)
        def _():
            for c in w_copies(0, 0):
                c.start()

        @pl.when(step + 1 < nsteps)
        def _():
            for c in w_copies(step + 1, 1 - slot):
                c.start()

        for c in w_copies(step, slot):
            c.wait()

        nb = meta_ref[0, e]
        poff = meta_ref[1, e]
        w1 = w1b[slot].astype(jnp.bfloat16)
        w3 = w3b[slot].astype(jnp.bfloat16)
        w2 = w2b[slot].astype(jnp.bfloat16)
        dn = (((1,), (1,)), ((), ()))

        def blk(i, carry):
            r0 = pl.multiple_of(poff + i * M, M)
            x = x_ref[pl.ds(r0, M), :].astype(jnp.bfloat16)
            gate = lax.dot_general(x, w1, dn,
                                   preferred_element_type=jnp.float32)
            up = lax.dot_general(x, w3, dn,
                                 preferred_element_type=jnp.float32)
            act = (gate * lax.logistic(gate) * up).astype(jnp.bfloat16)
            yv = lax.dot_general(act, w2, dn,
                                 preferred_element_type=jnp.float32)
            a0 = pl.multiple_of(i * M, M)

            @pl.when(f == 0)
            def _():
                acc[pl.ds(a0, M), :] = yv

            @pl.when(f != 0)
            def _():
                acc[pl.ds(a0, M), :] += yv

            @pl.when(f == F - 1)
            def _():
                cp = pltpu.make_async_copy(
                    acc.at[pl.ds(a0, M), :], y_ref.at[pl.ds(r0, M), :], osem)
                cp.start()
                cp.wait()

            return carry

        lax.fori_loop(0, nb, blk, 0)

    return _ffn_body


def _ffn(meta, x_s, w13_weight, w2_weight, F=4):
    S, H = x_s.shape
    E = w13_weight.shape[0]
    ff = w2_weight.shape[2]
    ffc = ff // F
    grid_spec = pltpu.PrefetchScalarGridSpec(
        num_scalar_prefetch=1,
        grid=(E * F,),
        in_specs=[
            pl.BlockSpec((S, H), lambda s, meta: (0, 0)),
            pl.BlockSpec(memory_space=pltpu.MemorySpace.HBM),
            pl.BlockSpec(memory_space=pltpu.MemorySpace.HBM),
        ],
        out_specs=pl.BlockSpec(memory_space=pltpu.MemorySpace.HBM),
        scratch_shapes=[
            pltpu.VMEM((2, ffc, H), jnp.float32),
            pltpu.VMEM((2, ffc, H), jnp.float32),
            pltpu.VMEM((2, H, ffc), jnp.float32),
            pltpu.VMEM((2048, H), jnp.float32),
            pltpu.SemaphoreType.DMA((2, 3)),
            pltpu.SemaphoreType.DMA,
        ],
    )
    return pl.pallas_call(
        _make_ffn_body(F, ffc, ff),
        grid_spec=grid_spec,
        out_shape=jax.ShapeDtypeStruct((S, H), jnp.float32),
        compiler_params=pltpu.CompilerParams(
            dimension_semantics=("arbitrary",),
            vmem_limit_bytes=100 * 1024 * 1024,
        ),
    )(meta, x_s, w13_weight, w2_weight)


def _combine(y_s, d0r, d1r, w0r, w1r, T, H):
    HALF = T // (NW * 2)
    mesh = plsc.VectorSubcoreMesh(core_axis_name="c", subcore_axis_name="s")

    @functools.partial(
        pl.kernel,
        out_type=jax.ShapeDtypeStruct((T, H), jnp.float32),
        mesh=mesh,
        scratch_types=[
            pltpu.VMEM((HALF, H), jnp.float32),
            pltpu.VMEM((HALF, H), jnp.float32),
            pltpu.VMEM((1, HALF), jnp.int32),
            pltpu.VMEM((1, HALF), jnp.int32),
            pltpu.VMEM((1, HALF), jnp.float32),
            pltpu.VMEM((1, HALF), jnp.float32),
            pltpu.SemaphoreType.DMA,
            pltpu.SemaphoreType.DMA,
        ],
    )
    def k(y_hbm, d0_hbm, d1_hbm, w0_hbm, w1_hbm, out_hbm,
          buf0, buf1, idx0, idx1, wb0, wb1, sem0, sem1):
        wid = lax.axis_index("s") * 2 + lax.axis_index("c")

        @pl.loop(0, 2)
        def _(h):
            chunk = wid * 2 + h
            base = chunk * HALF
            pltpu.sync_copy(d0_hbm.at[pl.ds(chunk, 1)], idx0)
            pltpu.sync_copy(d1_hbm.at[pl.ds(chunk, 1)], idx1)
            pltpu.sync_copy(w0_hbm.at[pl.ds(chunk, 1)], wb0)
            pltpu.sync_copy(w1_hbm.at[pl.ds(chunk, 1)], wb1)
            cp0 = pltpu.async_copy(y_hbm.at[idx0.at[0]], buf0, sem0)
            cp1 = pltpu.async_copy(y_hbm.at[idx1.at[0]], buf1, sem1)
            cp0.wait()
            cp1.wait()

            @pl.loop(0, HALF // L)
            def _(g):
                wv0 = wb0[0, pl.ds(g * L, L)]
                wv1 = wb1[0, pl.ds(g * L, L)]

                @pl.loop(0, L)
                def _(j):
                    r = g * L + j
                    jv = jnp.full((L,), j, jnp.int32)
                    w0v = wv0.at[jv].get(mode="promise_in_bounds")
                    w1v = wv1.at[jv].get(mode="promise_in_bounds")

                    for c in range(H // L):
                        a = buf0[r, pl.ds(c * L, L)]
                        b = buf1[r, pl.ds(c * L, L)]
                        buf0[r, pl.ds(c * L, L)] = a * w0v + b * w1v

            pltpu.sync_copy(buf0, out_hbm.at[pl.ds(base, HALF)])

    return k(y_s, d0r, d1r, w0r, w1r)


def kernel(hidden_states, router_logits, w13_weight, w2_weight):
    T, H = hidden_states.shape
    E = router_logits.shape[1]
    K = 2
    S = (T * K // M + E) * M

    dst, w01, meta = _routing(router_logits.T)

    TPW = T // NW
    dst0r = dst[0].reshape(NW, TPW)
    dst1r = dst[1].reshape(NW, TPW)

    x_s = _dispatch(hidden_states, dst0r, dst1r, S)
    y_s = _ffn(meta, x_s, w13_weight, w2_weight)

    HALF = T // (NW * 2)
    d0c = dst[0].reshape(NW * 2, HALF)
    d1c = dst[1].reshape(NW * 2, HALF)
    w0c = w01[0].reshape(NW * 2, HALF)
    w1c = w01[1].reshape(NW * 2, HALF)
    return _combine(y_s, d0c, d1c, w0c, w1c, T, H)
